# baseline (speedup 1.0000x reference)
import functools

import jax
import jax.numpy as jnp
from jax import lax
from jax.experimental import pallas as pl
from jax.experimental.pallas import tpu as pltpu
from jax.experimental.pallas import tpu_sc as plsc

MAX_POSITION = 8192
EMBED_DIM = 1024
BATCH = 4
SEQ_LEN = 8192

_INFO = plsc.get_sparse_core_info()
_NC = _INFO.num_cores
_NS = _INFO.num_subcores
_NW = _NC * _NS

_B = BATCH * SEQ_LEN
_B_PER_W = _B // _NW
_CHUNK = 8
_NBUF = 8
_LAG = 2

_STREAM_CHUNKS = 64
_STREAM_ROWS = _STREAM_CHUNKS * _CHUNK
_DMA_ROWS = _B_PER_W - _STREAM_ROWS
_N_CHUNKS = _STREAM_CHUNKS
_GROUPS = _STREAM_CHUNKS // _NBUF
_DBATCH = _DMA_ROWS // _GROUPS


def _gather_body(idx_hbm, table_hbm, out_hbm, idx_v, *scratch):
    wid = lax.axis_index("s") * _NC + lax.axis_index("c")
    base = wid * _B_PER_W
    pltpu.sync_copy(idx_hbm.at[pl.ds(base, _B_PER_W)], idx_v)

    rows = scratch[:_NBUF]
    gsem = scratch[_NBUF:2 * _NBUF]
    ssem = scratch[2 * _NBUF:3 * _NBUF]
    dsem = scratch[3 * _NBUF]
    idx_smem = scratch[3 * _NBUF + 1]
    idx_shared = scratch[3 * _NBUF + 2]
    sid = lax.axis_index("s")
    pltpu.sync_copy(idx_hbm.at[pl.ds(base + _STREAM_ROWS, _DMA_ROWS)],
                    idx_shared.at[sid])
    pltpu.sync_copy(idx_shared.at[sid], idx_smem)

    def gather_copy(i, buf):
        off = pl.multiple_of(i * _CHUNK, _CHUNK)
        return pltpu.make_async_copy(
            table_hbm.at[idx_v.at[pl.ds(off, _CHUNK)]], rows[buf], gsem[buf]
        )

    def store_copy(i, buf):
        off = pl.multiple_of(i * _CHUNK, _CHUNK)
        return pltpu.make_async_copy(
            rows[buf], out_hbm.at[pl.ds(base + off, _CHUNK)], ssem[buf]
        )

    def a_step(i, b):
        gather_copy(i, b).wait()
        store_copy(i, b).start()

    def b_step(i, b):
        store_copy(i, b).wait()
        gather_copy(i + _NBUF, b).start()

    def dma_row_copy(t):
        return pltpu.make_async_copy(
            table_hbm.at[pl.ds(idx_smem[t], 1)],
            out_hbm.at[pl.ds(base + _STREAM_ROWS + t, 1)],
            dsem,
        )

    def dma_batch(g):
        def issue(t, _):
            dma_row_copy(t).start()
            return 0
        lax.fori_loop(g * _DBATCH, (g + 1) * _DBATCH, issue, 0)

    for b in range(_NBUF):
        gather_copy(b, b).start()

    dma_batch(0)
    for b in range(_NBUF):
        a_step(b, b)
        if b >= _LAG:
            b_step(b - _LAG, b - _LAG)

    def group(g, _):
        dma_batch(g)
        cb = g * _NBUF
        for b in range(_NBUF):
            a_step(cb + b, b)
            k = cb + b - _LAG

            @pl.when(k + _NBUF < _N_CHUNKS)
            def _(k=k, b=b):
                b_step(k, (b - _LAG) % _NBUF)
        return 0

    lax.fori_loop(1, _N_CHUNKS // _NBUF, group, 0)
    for b in range(_NBUF):
        store_copy(_N_CHUNKS - _NBUF + b, b).wait()

    def drain(t, _):
        dma_row_copy(t).wait()
        return 0

    lax.fori_loop(0, _DMA_ROWS, drain, 0)


@jax.jit
def _embed_gather(position_ids_flat, table):
    mesh = plsc.VectorSubcoreMesh(core_axis_name="c", subcore_axis_name="s")
    kern = functools.partial(
        pl.kernel,
        mesh=mesh,
        out_type=jax.ShapeDtypeStruct((_B, EMBED_DIM), jnp.float32),
        scratch_types=(
            [pltpu.VMEM((_B_PER_W,), jnp.int32)]
            + [pltpu.VMEM((_CHUNK, EMBED_DIM), jnp.float32) for _ in range(_NBUF)]
            + [pltpu.SemaphoreType.DMA for _ in range(2 * _NBUF)]
            + [pltpu.SemaphoreType.DMA, pltpu.SMEM((_DMA_ROWS,), jnp.int32),
               pltpu.VMEM_SHARED((_NS, _DMA_ROWS), jnp.int32)]
        ),
    )(_gather_body)
    return kern(position_ids_flat, table)


def kernel(position_ids, table):
    flat = position_ids.reshape(-1).astype(jnp.int32)
    out = _embed_gather(flat, table)
    return out.reshape(BATCH, SEQ_LEN, EMBED_DIM)


# speedup vs baseline: 18.2257x; 18.2257x over previous
import functools

import jax
import jax.numpy as jnp
from jax import lax
from jax.experimental import pallas as pl
from jax.experimental.pallas import tpu as pltpu
from jax.experimental.pallas import tpu_sc as plsc

MAX_POSITION = 8192
EMBED_DIM = 1024
BATCH = 4
SEQ_LEN = 8192

_INFO = plsc.get_sparse_core_info()
_NC = _INFO.num_cores
_NS = _INFO.num_subcores
_NW = _NC * _NS

_B = BATCH * SEQ_LEN
_B_PER_W = _B // _NW
_CHUNK = 16
_NBUF = 4
_LAG = 2
_N_CHUNKS = _B_PER_W // _CHUNK


def _gather_body(idx_hbm, table_hbm, out_hbm, idx_v, *scratch):
    wid = lax.axis_index("s") * _NC + lax.axis_index("c")
    base = wid * _B_PER_W
    pltpu.sync_copy(idx_hbm.at[pl.ds(base, _B_PER_W)], idx_v)

    rows = scratch[:_NBUF]
    gsem = scratch[_NBUF:2 * _NBUF]
    ssem = scratch[2 * _NBUF:]

    def gather_copy(i, buf):
        off = pl.multiple_of(i * _CHUNK, _CHUNK)
        return pltpu.make_async_copy(
            table_hbm.at[idx_v.at[pl.ds(off, _CHUNK)]], rows[buf], gsem[buf]
        )

    def store_copy(i, buf):
        off = pl.multiple_of(i * _CHUNK, _CHUNK)
        return pltpu.make_async_copy(
            rows[buf], out_hbm.at[pl.ds(base + off, _CHUNK)], ssem[buf]
        )

    def a_step(i, b):
        gather_copy(i, b).wait()
        store_copy(i, b).start()

    def b_step(i, b):
        store_copy(i, b).wait()
        gather_copy(i + _NBUF, b).start()

    for b in range(_NBUF):
        gather_copy(b, b).start()

    for b in range(_NBUF):
        a_step(b, b)
        if b >= _LAG:
            b_step(b - _LAG, b - _LAG)

    def group(g, _):
        cb = g * _NBUF
        for b in range(_NBUF):
            a_step(cb + b, b)
            k = cb + b - _LAG

            @pl.when(k + _NBUF < _N_CHUNKS)
            def _(k=k, b=b):
                b_step(k, (b - _LAG) % _NBUF)
        return 0

    lax.fori_loop(1, _N_CHUNKS // _NBUF, group, 0)
    for b in range(_NBUF):
        store_copy(_N_CHUNKS - _NBUF + b, b).wait()


@jax.jit
def _embed_gather(position_ids_flat, table):
    mesh = plsc.VectorSubcoreMesh(core_axis_name="c", subcore_axis_name="s")
    kern = functools.partial(
        pl.kernel,
        mesh=mesh,
        out_type=jax.ShapeDtypeStruct((_B, EMBED_DIM), jnp.float32),
        scratch_types=(
            [pltpu.VMEM((_B_PER_W,), jnp.int32)]
            + [pltpu.VMEM((_CHUNK, EMBED_DIM), jnp.float32) for _ in range(_NBUF)]
            + [pltpu.SemaphoreType.DMA for _ in range(2 * _NBUF)]
        ),
    )(_gather_body)
    return kern(position_ids_flat, table)


def kernel(position_ids, table):
    flat = position_ids.reshape(-1).astype(jnp.int32)
    out = _embed_gather(flat, table)
    return out.reshape(BATCH, SEQ_LEN, EMBED_DIM)


# speedup vs baseline: 18.3529x; 1.0070x over previous
import functools

import jax
import jax.numpy as jnp
from jax import lax
from jax.experimental import pallas as pl
from jax.experimental.pallas import tpu as pltpu
from jax.experimental.pallas import tpu_sc as plsc

MAX_POSITION = 8192
EMBED_DIM = 1024
BATCH = 4
SEQ_LEN = 8192

_INFO = plsc.get_sparse_core_info()
_NC = _INFO.num_cores
_NS = _INFO.num_subcores
_NW = _NC * _NS

_B = BATCH * SEQ_LEN
_B_PER_W = _B // _NW
_CHUNK = 8
_NBUF = 8
_LAG = 2
_N_CHUNKS = _B_PER_W // _CHUNK


def _gather_body(idx_hbm, table_hbm, out_hbm, idx_v, *scratch):
    wid = lax.axis_index("s") * _NC + lax.axis_index("c")
    base = wid * _B_PER_W
    pltpu.sync_copy(idx_hbm.at[pl.ds(base, _B_PER_W)], idx_v)

    rows = scratch[:_NBUF]
    gsem = scratch[_NBUF:2 * _NBUF]
    ssem = scratch[2 * _NBUF:]

    def gather_copy(i, buf):
        off = pl.multiple_of(i * _CHUNK, _CHUNK)
        return pltpu.make_async_copy(
            table_hbm.at[idx_v.at[pl.ds(off, _CHUNK)]], rows[buf], gsem[buf]
        )

    def store_copy(i, buf):
        off = pl.multiple_of(i * _CHUNK, _CHUNK)
        return pltpu.make_async_copy(
            rows[buf], out_hbm.at[pl.ds(base + off, _CHUNK)], ssem[buf]
        )

    def a_step(i, b):
        gather_copy(i, b).wait()
        store_copy(i, b).start()

    def b_step(i, b):
        store_copy(i, b).wait()
        gather_copy(i + _NBUF, b).start()

    for b in range(_NBUF):
        gather_copy(b, b).start()

    for b in range(_NBUF):
        a_step(b, b)
        if b >= _LAG:
            b_step(b - _LAG, b - _LAG)

    def group(g, _):
        cb = g * _NBUF
        for b in range(_NBUF):
            a_step(cb + b, b)
            k = cb + b - _LAG

            @pl.when(k + _NBUF < _N_CHUNKS)
            def _(k=k, b=b):
                b_step(k, (b - _LAG) % _NBUF)
        return 0

    lax.fori_loop(1, _N_CHUNKS // _NBUF, group, 0)
    for b in range(_NBUF):
        store_copy(_N_CHUNKS - _NBUF + b, b).wait()


@jax.jit
def _embed_gather(position_ids_flat, table):
    mesh = plsc.VectorSubcoreMesh(core_axis_name="c", subcore_axis_name="s")
    kern = functools.partial(
        pl.kernel,
        mesh=mesh,
        out_type=jax.ShapeDtypeStruct((_B, EMBED_DIM), jnp.float32),
        scratch_types=(
            [pltpu.VMEM((_B_PER_W,), jnp.int32)]
            + [pltpu.VMEM((_CHUNK, EMBED_DIM), jnp.float32) for _ in range(_NBUF)]
            + [pltpu.SemaphoreType.DMA for _ in range(2 * _NBUF)]
        ),
    )(_gather_body)
    return kern(position_ids_flat, table)


def kernel(position_ids, table):
    flat = position_ids.reshape(-1).astype(jnp.int32)
    out = _embed_gather(flat, table)
    return out.reshape(BATCH, SEQ_LEN, EMBED_DIM)
